# baseline (speedup 1.0000x reference)
import functools

import jax
import jax.numpy as jnp
from jax import lax
from jax.experimental import pallas as pl
from jax.experimental.pallas import tpu as pltpu
from jax.experimental.pallas import tpu_sc as plsc

N = 10000
E = 320000
C = 128
CP = 144
NEG = 0.2

NC = 2
NS = 16
L = 16
NW = NC * NS

CHUNK = 64
SEG = 2048
NSEG0 = 6
NSEG1 = 4
CPS = SEG // CHUNK
TOT_SEG = NS * (NSEG0 + NSEG1)
E_PAD = SEG * TOT_SEG
N_PAD = 10240
RPS = N_PAD // NS


def _proj_body(x_ref, w_ref, emb_ref, atts_ref, xlp_ref, ai_ref, aj_ref):
    x = x_ref[...]
    xl = lax.dot_general(x, w_ref[...], (((1,), (1,)), ((), ())),
                         preferred_element_type=jnp.float32)
    emb = emb_ref[...]
    atts = atts_ref[...]
    ai_ref[...] = jnp.sum(xl * atts[0][None, :], axis=1) + \
        jnp.sum(emb * atts[2][None, :], axis=1)
    aj = jnp.sum(xl * atts[1][None, :], axis=1) + \
        jnp.sum(emb * atts[3][None, :], axis=1)
    aj_ref[...] = aj
    xlp_ref[...] = jnp.concatenate([
        xl,
        jnp.ones((N, 1), jnp.float32),
        aj[:, None],
        jnp.zeros((N, CP - C - 2), jnp.float32),
    ], axis=1)


_proj = pl.pallas_call(
    _proj_body,
    out_shape=[
        jax.ShapeDtypeStruct((N, CP), jnp.float32),
        jax.ShapeDtypeStruct((N,), jnp.float32),
        jax.ShapeDtypeStruct((N,), jnp.float32),
    ],
)


_mesh = plsc.VectorSubcoreMesh(core_axis_name="c", subcore_axis_name="s",
                               num_cores=NC, num_subcores=NS)


@functools.partial(
    pl.kernel,
    out_type=jax.ShapeDtypeStruct((NC, N_PAD, CP), jnp.float32),
    mesh=_mesh,
    compiler_params=pltpu.CompilerParams(needs_layout_passes=False,
                                         use_tc_tiling_on_sc=False),
    scratch_types=[
        pltpu.VMEM((N,), jnp.float32),
        pltpu.VMEM((SEG,), jnp.int32),
        pltpu.VMEM((CPS, CHUNK), jnp.int32),
        pltpu.VMEM((CHUNK,), jnp.float32),
        pltpu.VMEM((CHUNK, CP), jnp.float32),
        pltpu.VMEM((CHUNK, CP), jnp.float32),
        pltpu.VMEM_SHARED((N_PAD, CP), jnp.float32),
        pltpu.SemaphoreType.DMA,
        pltpu.SemaphoreType.DMA,
    ],
)
def _edge_kernel(src_hbm, dst_hbm, ai_hbm, xlp_hbm, outp_hbm,
                 ai_l, src_seg, dst_seg, w_buf, rows_a, rows_b,
                 out_sh, sem_a, sem_b):
    cid = lax.axis_index("c")
    sid = lax.axis_index("s")
    nseg = jnp.where(cid == 0, NSEG0, NSEG1)
    seg_base = jnp.where(cid == 0, sid * NSEG0, NS * NSEG0 + sid * NSEG1)

    zero16 = jnp.zeros((L,), jnp.float32)

    def zrow(i, _):
        for k8 in range(CP // L):
            rows_a[i, pl.ds(k8 * L, L)] = zero16
        return 0
    lax.fori_loop(0, CHUNK, zrow, 0)
    base_n = sid * RPS
    for j in range(RPS // CHUNK):
        pltpu.sync_copy(rows_a, out_sh.at[pl.ds(base_n + j * CHUNK, CHUNK)])

    pltpu.sync_copy(ai_hbm, ai_l)
    plsc.subcore_barrier()

    lane = lax.iota(jnp.int32, 16)

    def gather(c, buf, sem):
        pltpu.async_copy(xlp_hbm.at[src_seg.at[pl.ds(c * CHUNK, CHUNK)]],
                         buf, sem)

    def stage(c, buf, sem, obuf, osem):
        pltpu.make_async_copy(
            xlp_hbm.at[src_seg.at[pl.ds(c * CHUNK, CHUNK)]], buf, sem).wait()

        @pl.when(c + 1 < CPS)
        def _():
            gather(c + 1, obuf, osem)

        def wbody(k, _):
            s16 = src_seg[pl.ds(c * CHUNK + k * L, L)]
            d16 = dst_seg[c, pl.ds(k * L, L)]
            aj16 = plsc.load_gather(buf, [k * L + lane,
                                          jnp.full((L,), C + 1, jnp.int32)])
            a = plsc.load_gather(ai_l, [d16]) + aj16
            a = jnp.where(a >= 0, a, a * jnp.float32(NEG))
            w = jnp.where(s16 == d16, jnp.float32(0.0), jnp.exp(a))
            w_buf[pl.ds(k * L, L)] = w
            return 0
        lax.fori_loop(0, CHUNK // L, wbody, 0)

        def mbody(i, _):
            wb = plsc.load_gather(w_buf, [jnp.full((L,), i, jnp.int32)])
            for k8 in range(CP // L):
                buf[i, pl.ds(k8 * L, L)] = buf[i, pl.ds(k8 * L, L)] * wb
            return 0
        lax.fori_loop(0, CHUNK, mbody, 0)

        pltpu.sync_copy(buf, out_sh.at[dst_seg.at[c]], add=True)

    def seg_body(g, _):
        pltpu.sync_copy(src_hbm.at[seg_base + g], src_seg)
        pltpu.sync_copy(dst_hbm.at[seg_base + g], dst_seg)
        gather(0, rows_a, sem_a)

        def pipe(tt, _):
            stage(tt * 2, rows_a, sem_a, rows_b, sem_b)
            stage(tt * 2 + 1, rows_b, sem_b, rows_a, sem_a)
            return 0
        lax.fori_loop(0, CPS // 2, pipe, 0)
        return 0
    lax.fori_loop(0, nseg, seg_body, 0)

    plsc.subcore_barrier()
    for j in range(RPS // CHUNK):
        sl = pl.ds(base_n + j * CHUNK, CHUNK)
        pltpu.sync_copy(out_sh.at[sl], outp_hbm.at[cid, sl])


def _final_body(outp_ref, xlp_ref, ai_ref, aj_ref,
                bias_ref, g_ref, b_ref, o_ref):
    al = ai_ref[...] + aj_ref[...]
    wl = jnp.exp(jnp.where(al >= 0, al, al * jnp.float32(NEG)))
    xl = xlp_ref[:, :C]
    num = outp_ref[0, :N, :C] + outp_ref[1, :N, :C] + wl[:, None] * xl
    den = outp_ref[0, :N, C] + outp_ref[1, :N, C] + wl + jnp.float32(1e-16)
    o = num / den[:, None] + bias_ref[...][None, :]
    mu = jnp.mean(o, axis=0)
    var = jnp.mean((o - mu[None, :]) ** 2, axis=0)
    o_ref[...] = (o - mu[None, :]) * lax.rsqrt(var + jnp.float32(1e-5)) \
        * g_ref[...][None, :] + b_ref[...][None, :]


_final = pl.pallas_call(
    _final_body,
    out_shape=jax.ShapeDtypeStruct((N, C), jnp.float32),
)


def kernel(x, edge_index, embedding, W, att_i, att_j, att_em_i, att_em_j,
           bias, bn_gamma, bn_beta):
    atts = jnp.concatenate([
        att_i.reshape(1, C), att_j.reshape(1, C),
        att_em_i.reshape(1, C), att_em_j.reshape(1, C)], axis=0)
    xlp, ai, aj = _proj(x, W, embedding, atts)

    pad = jnp.zeros((E_PAD - E,), jnp.int32)
    srcp = jnp.concatenate([edge_index[0], pad]).reshape(TOT_SEG, SEG)
    dstp = jnp.concatenate([edge_index[1], pad]).reshape(TOT_SEG, CPS, CHUNK)

    outp = _edge_kernel(srcp, dstp, ai, xlp)
    return _final(outp, xlp, ai, aj, bias, bn_gamma, bn_beta)


# speedup vs baseline: 1.1287x; 1.1287x over previous
import functools

import jax
import jax.numpy as jnp
from jax import lax
from jax.experimental import pallas as pl
from jax.experimental.pallas import tpu as pltpu
from jax.experimental.pallas import tpu_sc as plsc

N = 10000
E = 320000
C = 128
CP = 144
NEG = 0.2

NC = 2
NS = 16
L = 16
NW = NC * NS

CHUNK = 64
SEG = 2048
NSEG0 = 8
NSEG1 = 2
CPS = SEG // CHUNK
TOT_SEG = NS * (NSEG0 + NSEG1)
E_PAD = SEG * TOT_SEG
N_PAD = 10240
RPS = N_PAD // NS


def _proj_body(x_ref, w_ref, emb_ref, atts_ref, xlp_ref, ai_ref, aj_ref):
    x = x_ref[...]
    xl = lax.dot_general(x, w_ref[...], (((1,), (1,)), ((), ())),
                         preferred_element_type=jnp.float32)
    emb = emb_ref[...]
    atts = atts_ref[...]
    ai_ref[...] = jnp.sum(xl * atts[0][None, :], axis=1) + \
        jnp.sum(emb * atts[2][None, :], axis=1)
    aj = jnp.sum(xl * atts[1][None, :], axis=1) + \
        jnp.sum(emb * atts[3][None, :], axis=1)
    aj_ref[...] = aj
    xlp_ref[...] = jnp.concatenate([
        xl,
        jnp.ones((N, 1), jnp.float32),
        aj[:, None],
        jnp.zeros((N, CP - C - 2), jnp.float32),
    ], axis=1)


_proj = pl.pallas_call(
    _proj_body,
    out_shape=[
        jax.ShapeDtypeStruct((N, CP), jnp.float32),
        jax.ShapeDtypeStruct((N,), jnp.float32),
        jax.ShapeDtypeStruct((N,), jnp.float32),
    ],
)


_mesh = plsc.VectorSubcoreMesh(core_axis_name="c", subcore_axis_name="s",
                               num_cores=NC, num_subcores=NS)


@functools.partial(
    pl.kernel,
    out_type=jax.ShapeDtypeStruct((NC, N_PAD, CP), jnp.float32),
    mesh=_mesh,
    compiler_params=pltpu.CompilerParams(needs_layout_passes=False,
                                         use_tc_tiling_on_sc=False),
    scratch_types=[
        pltpu.VMEM((N,), jnp.float32),
        pltpu.VMEM((SEG,), jnp.int32),
        pltpu.VMEM((CPS, CHUNK), jnp.int32),
        pltpu.VMEM((CHUNK,), jnp.float32),
        pltpu.VMEM((CHUNK, CP), jnp.float32),
        pltpu.VMEM((CHUNK, CP), jnp.float32),
        pltpu.VMEM_SHARED((N_PAD, CP), jnp.float32),
        pltpu.SemaphoreType.DMA,
        pltpu.SemaphoreType.DMA,
    ],
)
def _edge_kernel(src_hbm, dst_hbm, ai_hbm, xlp_hbm, outp_hbm,
                 ai_l, src_seg, dst_seg, w_buf, rows_a, rows_b,
                 out_sh, sem_a, sem_b):
    cid = lax.axis_index("c")
    sid = lax.axis_index("s")
    nseg = jnp.where(cid == 0, NSEG0, NSEG1)
    seg_base = jnp.where(cid == 0, sid * NSEG0, NS * NSEG0 + sid * NSEG1)

    zero16 = jnp.zeros((L,), jnp.float32)

    def zrow(i, _):
        for k8 in range(CP // L):
            rows_a[i, pl.ds(k8 * L, L)] = zero16
        return 0
    lax.fori_loop(0, CHUNK, zrow, 0)
    base_n = sid * RPS
    for j in range(RPS // CHUNK):
        pltpu.sync_copy(rows_a, out_sh.at[pl.ds(base_n + j * CHUNK, CHUNK)])

    pltpu.sync_copy(ai_hbm, ai_l)
    plsc.subcore_barrier()

    lane = lax.iota(jnp.int32, 16)

    def gather(c, buf, sem):
        pltpu.async_copy(xlp_hbm.at[src_seg.at[pl.ds(c * CHUNK, CHUNK)]],
                         buf, sem)

    def stage(c, buf, sem, obuf, osem):
        pltpu.make_async_copy(
            xlp_hbm.at[src_seg.at[pl.ds(c * CHUNK, CHUNK)]], buf, sem).wait()

        @pl.when(c + 1 < CPS)
        def _():
            gather(c + 1, obuf, osem)

        def wbody(k, _):
            s16 = src_seg[pl.ds(c * CHUNK + k * L, L)]
            d16 = dst_seg[c, pl.ds(k * L, L)]
            aj16 = plsc.load_gather(buf, [k * L + lane,
                                          jnp.full((L,), C + 1, jnp.int32)])
            a = plsc.load_gather(ai_l, [d16]) + aj16
            a = jnp.where(a >= 0, a, a * jnp.float32(NEG))
            w = jnp.where(s16 == d16, jnp.float32(0.0), jnp.exp(a))
            w_buf[pl.ds(k * L, L)] = w
            return 0
        lax.fori_loop(0, CHUNK // L, wbody, 0)

        def mbody(i, _):
            wb = plsc.load_gather(w_buf, [jnp.full((L,), i, jnp.int32)])
            for k8 in range(CP // L):
                buf[i, pl.ds(k8 * L, L)] = buf[i, pl.ds(k8 * L, L)] * wb
            return 0
        lax.fori_loop(0, CHUNK, mbody, 0)

        pltpu.sync_copy(buf, out_sh.at[dst_seg.at[c]], add=True)

    def seg_body(g, _):
        pltpu.sync_copy(src_hbm.at[seg_base + g], src_seg)
        pltpu.sync_copy(dst_hbm.at[seg_base + g], dst_seg)
        gather(0, rows_a, sem_a)

        def pipe(tt, _):
            stage(tt * 2, rows_a, sem_a, rows_b, sem_b)
            stage(tt * 2 + 1, rows_b, sem_b, rows_a, sem_a)
            return 0
        lax.fori_loop(0, CPS // 2, pipe, 0)
        return 0
    lax.fori_loop(0, nseg, seg_body, 0)

    plsc.subcore_barrier()
    for j in range(RPS // CHUNK):
        sl = pl.ds(base_n + j * CHUNK, CHUNK)
        pltpu.sync_copy(out_sh.at[sl], outp_hbm.at[cid, sl])


def _final_body(outp_ref, xlp_ref, ai_ref, aj_ref,
                bias_ref, g_ref, b_ref, o_ref):
    al = ai_ref[...] + aj_ref[...]
    wl = jnp.exp(jnp.where(al >= 0, al, al * jnp.float32(NEG)))
    xl = xlp_ref[:, :C]
    num = outp_ref[0, :N, :C] + outp_ref[1, :N, :C] + wl[:, None] * xl
    den = outp_ref[0, :N, C] + outp_ref[1, :N, C] + wl + jnp.float32(1e-16)
    o = num / den[:, None] + bias_ref[...][None, :]
    mu = jnp.mean(o, axis=0)
    var = jnp.mean((o - mu[None, :]) ** 2, axis=0)
    o_ref[...] = (o - mu[None, :]) * lax.rsqrt(var + jnp.float32(1e-5)) \
        * g_ref[...][None, :] + b_ref[...][None, :]


_final = pl.pallas_call(
    _final_body,
    out_shape=jax.ShapeDtypeStruct((N, C), jnp.float32),
)


def kernel(x, edge_index, embedding, W, att_i, att_j, att_em_i, att_em_j,
           bias, bn_gamma, bn_beta):
    atts = jnp.concatenate([
        att_i.reshape(1, C), att_j.reshape(1, C),
        att_em_i.reshape(1, C), att_em_j.reshape(1, C)], axis=0)
    xlp, ai, aj = _proj(x, W, embedding, atts)

    pad = jnp.zeros((E_PAD - E,), jnp.int32)
    srcp = jnp.concatenate([edge_index[0], pad]).reshape(TOT_SEG, SEG)
    dstp = jnp.concatenate([edge_index[1], pad]).reshape(TOT_SEG, CPS, CHUNK)

    outp = _edge_kernel(srcp, dstp, ai, xlp)
    return _final(outp, xlp, ai, aj, bias, bn_gamma, bn_beta)
